# Initial kernel scaffold; baseline (speedup 1.0000x reference)
import jax
import jax.numpy as jnp
from jax.experimental import pallas as pl


def kernel(feats_A, pca_C2, pca_C3, params, edge_bond, edge_knn, g1_dst, edge_I2, g2_dst, edge_I3):
    raise NotImplementedError("write your pallas kernel here")


# speedup vs baseline: 18.2434x; 18.2434x over previous
import functools

import jax
import jax.numpy as jnp
from jax import lax
from jax.experimental import pallas as pl
from jax.experimental.pallas import tpu as pltpu
from jax.experimental.pallas import tpu_sc as plsc

NA = 10000
IN = 128
HID = 128
HEADS = 4
NC2 = 2000
NC3 = 400
EB = 160000
EI2 = 32000
EI3 = 6400

NCORE = 2
NSUB = 16
NWORK = NCORE * NSUB
CH = 100

_f32 = jnp.float32


def _mesh():
    return plsc.VectorSubcoreMesh(
        core_axis_name="c", subcore_axis_name="s",
        num_cores=NCORE, num_subcores=NSUB)


def _row_split_copy(src, dst, s, n):
    rz = (n // NSUB // 8) * 8
    tail = n - rz * NSUB
    pltpu.sync_copy(src.at[pl.ds(s * rz, rz)], dst.at[pl.ds(s * rz, rz)])
    if tail:
        @pl.when(s == NSUB - 1)
        def _():
            pltpu.sync_copy(src.at[pl.ds(NSUB * rz, tail)],
                            dst.at[pl.ds(NSUB * rz, tail)])


def _sc_seg_sum(table, src3, dst3, zrows, n, d, e):
    K = d // 128
    ec = e // NWORK
    nch = ec // CH

    @functools.partial(
        pl.kernel,
        out_type=jax.ShapeDtypeStruct((NCORE, K, n, HID), _f32),
        mesh=_mesh(),
        scratch_types=[
            pltpu.VMEM((nch, CH), jnp.int32),
            pltpu.VMEM((nch, CH), jnp.int32),
            pltpu.VMEM((CH, HID), _f32),
            pltpu.VMEM_SHARED((K, n, HID), _f32),
            pltpu.SemaphoreType.DMA,
        ],
    )
    def k(tbl, sr, ds_, zr, out, src_v, dst_v, rows_v, acc, sem):
        c = lax.axis_index("c")
        s = lax.axis_index("s")
        w = c * NSUB + s
        for kk in range(K):
            _row_split_copy(zr, acc.at[kk], s, n)
        pltpu.sync_copy(sr.at[w], src_v)
        pltpu.sync_copy(ds_.at[w], dst_v)
        plsc.subcore_barrier()

        def body(j, carry):
            for kk in range(K):
                pltpu.async_copy(tbl.at[kk].at[src_v.at[j]], rows_v,
                                 sem).wait()
                pltpu.sync_copy(rows_v, acc.at[kk].at[dst_v.at[j]], add=True)
            return carry

        lax.fori_loop(0, nch, body, 0)
        plsc.subcore_barrier()
        for kk in range(K):
            _row_split_copy(acc.at[kk], out.at[c, kk], s, n)

    return k(table, src3, dst3, zrows)


NA8 = NA // 8
CHG = 100


def _sc_gat(g_tbl, er_tbl, src3, dst3, zs):
    ec = EB // NWORK
    nch = ec // CHG
    H = CHG // 2
    tail = H % 16

    @functools.partial(
        pl.kernel,
        out_type=(jax.ShapeDtypeStruct((NCORE, NA, HID), _f32),
                  jax.ShapeDtypeStruct((NWORK, ec // H, H, 16), _f32)),
        mesh=_mesh(),
        scratch_types=[
            pltpu.VMEM((nch, CHG), jnp.int32),
            pltpu.VMEM((nch, CHG), jnp.int32),
            pltpu.VMEM((H, 2 * HID), _f32),
            pltpu.VMEM((H, HID), _f32),
            pltpu.VMEM((H, HID), _f32),
            pltpu.VMEM((H, 16), _f32),
            pltpu.VMEM_SHARED((NA, HID), _f32),
            pltpu.SemaphoreType.DMA,
        ],
    )
    def k(gt, et, sr, ds_, zs_h, s_out, ex_out,
          src_v, dst_v, gb, eb, wb, exb, s_acc, sem):
        c = lax.axis_index("c")
        s = lax.axis_index("s")
        w = c * NSUB + s
        _row_split_copy(zs_h, s_acc, s, NA)
        pltpu.sync_copy(sr.at[w], src_v)
        pltpu.sync_copy(ds_.at[w], dst_v)
        plsc.subcore_barrier()

        def do_edge(i):
            elv = gb[i, pl.ds(HID, 16)]
            erv = eb[i, pl.ds(0, 16)]
            t = elv + erv
            ex = jnp.exp(jnp.maximum(t, 0.2 * t))
            exb[i] = ex
            for kk in range(8):
                exh = jnp.full((16,), ex[kk // 2], _f32)
                wb[i, pl.ds(kk * 16, 16)] = gb[i, pl.ds(kk * 16, 16)] * exh

        def half(j, h):
            sidx = src_v.at[j, pl.ds(h * H, H)]
            didx = dst_v.at[j, pl.ds(h * H, H)]
            pltpu.async_copy(gt.at[sidx], gb, sem).wait()
            pltpu.async_copy(et.at[didx], eb, sem).wait()

            def grp(b, cc):
                for kk in range(16):
                    do_edge(b * 16 + kk)
                return cc

            lax.fori_loop(0, H // 16, grp, 0)
            if tail:
                for kk in range(16 - tail, 16):
                    do_edge(H - 16 + kk)
            pltpu.sync_copy(wb, s_acc.at[didx], add=True)
            pltpu.sync_copy(exb, ex_out.at[w].at[2 * j + h])

        def chunk(j, carry):
            half(j, 0)
            half(j, 1)
            return carry

        lax.fori_loop(0, nch, chunk, 0)
        plsc.subcore_barrier()
        _row_split_copy(s_acc, s_out.at[c], s, NA)

    return k(g_tbl, er_tbl, src3, dst3, zs)


def _sc_den(ex_e, dst3, d8_3, zden):
    ec = EB // NWORK
    nch = ec // CHG
    tail = CHG % 16

    @functools.partial(
        pl.kernel,
        out_type=jax.ShapeDtypeStruct((NCORE, NA8, HID), _f32),
        mesh=_mesh(),
        scratch_types=[
            pltpu.VMEM((nch, CHG), jnp.int32),
            pltpu.VMEM((nch, CHG), jnp.int32),
            pltpu.VMEM((CHG, 16), _f32),
            pltpu.VMEM((CHG, HID), _f32),
            pltpu.VMEM_SHARED((NA8, HID), _f32),
            pltpu.SemaphoreType.DMA,
        ],
    )
    def k(exh_h, ds_, d8, zd_h, den_out,
          dst_v, d8_v, exb, db, d_acc, sem):
        c = lax.axis_index("c")
        s = lax.axis_index("s")
        w = c * NSUB + s
        _row_split_copy(zd_h, d_acc, s, NA8)
        pltpu.sync_copy(ds_.at[w], dst_v)
        pltpu.sync_copy(d8.at[w], d8_v)
        plsc.subcore_barrier()

        def do_edge(i, d, d8s):
            ex = exb[i]
            dm = d - d8s * 8
            zero = jnp.zeros((16,), _f32)
            for g in range(8):
                db[i, pl.ds(g * 16, 16)] = jnp.where(dm == g, ex, zero)

        def chunk(j, carry):
            pltpu.sync_copy(exh_h.at[w].at[j], exb)

            def grp(b, cc):
                dvv = dst_v[j, pl.ds(b * 16, 16)]
                dv8 = d8_v[j, pl.ds(b * 16, 16)]
                for kk in range(16):
                    do_edge(b * 16 + kk, dvv[kk], dv8[kk])
                return cc

            lax.fori_loop(0, CHG // 16, grp, 0)
            if tail:
                dvv = dst_v[j, pl.ds(CHG - 16, 16)]
                dv8 = d8_v[j, pl.ds(CHG - 16, 16)]
                for kk in range(16 - tail, 16):
                    do_edge(CHG - 16 + kk, dvv[kk], dv8[kk])
            pltpu.sync_copy(db, d_acc.at[d8_v.at[j]], add=True)
            return carry

        lax.fori_loop(0, nch, chunk, 0)
        plsc.subcore_barrier()
        _row_split_copy(d_acc, den_out.at[c], s, NA8)

    return k(ex_e, dst3, d8_3, zden)


def _sc_seg_max(table, seg, n, nseg):
    npad = ((n + 63) // 64) * 64
    n4 = npad // 4
    table_t = jnp.pad(table, ((0, npad - n), (0, 0))).reshape(
        npad, 8, 16).transpose(1, 0, 2).reshape(8, npad * 16)
    seg4 = jnp.pad(seg.astype(jnp.int32), (0, npad - n)).reshape(4, n4)

    @functools.partial(
        pl.kernel,
        out_type=jax.ShapeDtypeStruct((NCORE, 2, 8, nseg * 16), _f32),
        mesh=_mesh(),
        scratch_types=[
            pltpu.VMEM((nseg * 16,), _f32),
            pltpu.VMEM((n4 * 16,), _f32),
            pltpu.VMEM((n4,), jnp.int32),
            pltpu.SemaphoreType.DMA,
        ],
    )
    def k(tbl, sg, out, acc, rowb, seg_v, sem):
        c = lax.axis_index("c")
        s = lax.axis_index("s")
        g = s // 2
        q = s % 2
        r0 = (2 * c + q) * n4
        pltpu.sync_copy(sg.at[2 * c + q], seg_v)
        pltpu.sync_copy(tbl.at[g, pl.ds(r0 * 16, n4 * 16)], rowb)

        def zr(r, cc):
            acc[pl.ds(r * 16, 16)] = jnp.zeros((16,), _f32)
            return cc

        lax.fori_loop(0, nseg, zr, 0)

        def row16(cb, cc):
            sv = seg_v[pl.ds(cb * 16, 16)]
            for kk in range(16):
                off = sv[kk] * 16
                acc[pl.ds(off, 16)] = jnp.maximum(
                    acc[pl.ds(off, 16)], rowb[pl.ds((cb * 16 + kk) * 16, 16)])
            return cc

        lax.fori_loop(0, n4 // 16, row16, 0)
        pltpu.sync_copy(acc, out.at[c, q, g])

    return k(table_t, seg4)


def _tc_zelter(x, wg, al16, ar16, b16):
    br = 1000
    grid = NA // br

    def body(x_r, wg_r, al_r, ar_r, b_r, g_r, er_r):
        z = jnp.dot(x_r[...], wg_r[...], preferred_element_type=_f32)
        el16 = jnp.dot(z, al_r[...], preferred_element_type=_f32) + b_r[...]
        pad = jnp.zeros((br, HID - 16), _f32)
        g_r[...] = jnp.concatenate([z, el16, pad], axis=1)
        er16 = jnp.dot(z, ar_r[...], preferred_element_type=_f32)
        er_r[...] = jnp.concatenate([er16, pad], axis=1)

    return pl.pallas_call(
        body,
        grid=(grid,),
        in_specs=[
            pl.BlockSpec((br, IN), lambda i: (i, 0)),
            pl.BlockSpec((IN, HID), lambda i: (0, 0)),
            pl.BlockSpec((HID, 16), lambda i: (0, 0)),
            pl.BlockSpec((HID, 16), lambda i: (0, 0)),
            pl.BlockSpec((1, 16), lambda i: (0, 0)),
        ],
        out_specs=[
            pl.BlockSpec((br, 2 * HID), lambda i: (i, 0)),
            pl.BlockSpec((br, HID), lambda i: (i, 0)),
        ],
        out_shape=[
            jax.ShapeDtypeStruct((NA, 2 * HID), _f32),
            jax.ShapeDtypeStruct((NA, HID), _f32),
        ],
    )(x, wg, al16, ar16, b16)


def _tc_gat_finish(s_p, den16, r16):
    br = 1000
    grid = NA // br

    def body(s_r, d_r, r_r, o_r):
        ssum = s_r[0] + s_r[1]
        dsum = d_r[0] + d_r[1]
        dex = jnp.dot(dsum, r_r[...], preferred_element_type=_f32)
        o_r[...] = jnp.maximum(ssum / (dex + 1e-9), 0.0)

    return pl.pallas_call(
        body,
        grid=(grid,),
        in_specs=[
            pl.BlockSpec((2, br, HID), lambda i: (0, i, 0)),
            pl.BlockSpec((2, br, 16), lambda i: (0, i, 0)),
            pl.BlockSpec((16, HID), lambda i: (0, 0)),
        ],
        out_specs=pl.BlockSpec((br, HID), lambda i: (i, 0)),
        out_shape=jax.ShapeDtypeStruct((NA, HID), _f32),
    )(s_p, den16, r16)


def _stats_update(st_r, h, first):
    @pl.when(first)
    def _():
        st_r[...] = jnp.zeros_like(st_r)

    su = jnp.sum(h, axis=0, keepdims=True)
    sq = jnp.sum(h * h, axis=0, keepdims=True)
    st_r[...] += jnp.concatenate(
        [su, sq, jnp.zeros((6, HID), _f32)], axis=0)


def _tc_a2_atoms(gat, x, az, ax, w1a, w1b, b1):
    br = 1000
    grid = NA // br

    def body(g_r, x_r, az_r, ax_r, wa_r, wb_r, b_r, h_r, st_r):
        xa = g_r[...] + az_r[0] + az_r[1]
        xb = x_r[...] + ax_r[0] + ax_r[1]
        h = (jnp.dot(xa, wa_r[...], preferred_element_type=_f32)
             + jnp.dot(xb, wb_r[...], preferred_element_type=_f32) + b_r[...])
        h_r[...] = h
        _stats_update(st_r, h, pl.program_id(0) == 0)

    return pl.pallas_call(
        body,
        grid=(grid,),
        in_specs=[
            pl.BlockSpec((br, HID), lambda i: (i, 0)),
            pl.BlockSpec((br, IN), lambda i: (i, 0)),
            pl.BlockSpec((2, br, HID), lambda i: (0, i, 0)),
            pl.BlockSpec((2, br, IN), lambda i: (0, i, 0)),
            pl.BlockSpec((HID, HID), lambda i: (0, 0)),
            pl.BlockSpec((IN, HID), lambda i: (0, 0)),
            pl.BlockSpec((1, HID), lambda i: (0, 0)),
        ],
        out_specs=[
            pl.BlockSpec((br, HID), lambda i: (i, 0)),
            pl.BlockSpec((8, HID), lambda i: (0, 0)),
        ],
        out_shape=[
            jax.ShapeDtypeStruct((NA, HID), _f32),
            jax.ShapeDtypeStruct((8, HID), _f32),
        ],
    )(gat, x, az, ax, w1a, w1b, b1)


def _tc_a2_gen(x, a_p, w1, b1, n, d):
    br = min(n, 1000)
    grid = n // br
    K = d // 128

    def body(x_r, a_r, w_r, b_r, h_r, st_r):
        xin = jnp.concatenate(
            [x_r[kk] + a_r[0, kk] + a_r[1, kk] for kk in range(K)], axis=1)
        h = jnp.dot(xin, w_r[...], preferred_element_type=_f32) + b_r[...]
        h_r[...] = h
        _stats_update(st_r, h, pl.program_id(0) == 0)

    return pl.pallas_call(
        body,
        grid=(grid,),
        in_specs=[
            pl.BlockSpec((K, br, 128), lambda i: (0, i, 0)),
            pl.BlockSpec((2, K, br, 128), lambda i: (0, 0, i, 0)),
            pl.BlockSpec((d, HID), lambda i: (0, 0)),
            pl.BlockSpec((1, HID), lambda i: (0, 0)),
        ],
        out_specs=[
            pl.BlockSpec((br, HID), lambda i: (i, 0)),
            pl.BlockSpec((8, HID), lambda i: (0, 0)),
        ],
        out_shape=[
            jax.ShapeDtypeStruct((n, HID), _f32),
            jax.ShapeDtypeStruct((8, HID), _f32),
        ],
    )(x, a_p, w1, b1)


def _tc_a3(h1, st, g, bt, w2, b2, n):
    br = min(n, 1000)
    grid = n // br
    inv_n = 1.0 / n

    def body(h_r, st_r, g_r, bt_r, w2_r, b2_r, o_r, cs_r):
        stv = st_r[...]
        mu = stv[0:1] * inv_n
        var = stv[1:2] * inv_n - mu * mu
        sc = g_r[...] * lax.rsqrt(var + 1e-5)
        hn = (h_r[...] - mu) * sc + bt_r[...]
        o = jnp.dot(jnp.maximum(hn, 0.0), w2_r[...],
                    preferred_element_type=_f32) + b2_r[...]
        o = jnp.maximum(o, 0.0)
        o_r[...] = o

        @pl.when(pl.program_id(0) == 0)
        def _():
            cs_r[...] = jnp.zeros_like(cs_r)

        cs_r[...] += jnp.concatenate(
            [jnp.sum(o, axis=0, keepdims=True), jnp.zeros((7, HID), _f32)], axis=0)

    return pl.pallas_call(
        body,
        grid=(grid,),
        in_specs=[
            pl.BlockSpec((br, HID), lambda i: (i, 0)),
            pl.BlockSpec((8, HID), lambda i: (0, 0)),
            pl.BlockSpec((1, HID), lambda i: (0, 0)),
            pl.BlockSpec((1, HID), lambda i: (0, 0)),
            pl.BlockSpec((HID, HID), lambda i: (0, 0)),
            pl.BlockSpec((1, HID), lambda i: (0, 0)),
        ],
        out_specs=[
            pl.BlockSpec((br, HID), lambda i: (i, 0)),
            pl.BlockSpec((8, HID), lambda i: (0, 0)),
        ],
        out_shape=[
            jax.ShapeDtypeStruct((n, HID), _f32),
            jax.ShapeDtypeStruct((8, HID), _f32),
        ],
    )(h1, st, g, bt, w2, b2)


def _tc_combine(pairs, pca, n, dout):
    br = min(n, 1000)
    grid = n // br
    npair = len(pairs)
    K = dout // 128
    dpad = dout - 128 * npair - 16

    def body(*refs):
        prs = refs[:npair]
        p_r = refs[npair]
        o_r = refs[npair + 1]
        for kk, m in enumerate(prs):
            o_r[kk] = jnp.maximum(jnp.maximum(m[0], m[1]),
                                  jnp.maximum(m[2], m[3]))
        o_r[npair] = jnp.concatenate(
            [p_r[...], jnp.zeros((br, dpad), _f32)], axis=1)

    in_specs = [pl.BlockSpec((4, br, 128), lambda i: (0, i, 0))
                for _ in pairs]
    in_specs.append(pl.BlockSpec((br, 16), lambda i: (i, 0)))
    return pl.pallas_call(
        body,
        grid=(grid,),
        in_specs=in_specs,
        out_specs=pl.BlockSpec((K, br, 128), lambda i: (0, i, 0)),
        out_shape=jax.ShapeDtypeStruct((K, n, 128), _f32),
    )(*pairs, pca)


def _tc_readout(csb, csk, cs2, cs3, w1, b1, w2t, b2p):
    def body(cb, ck, c2, c3, w1_r, b1_r, w2_r, b2_r, o_r):
        r = jnp.concatenate(
            [cb[0:1] * (1.0 / NA), ck[0:1] * (1.0 / NA),
             c2[0:1] * (1.0 / NC2), c3[0:1] * (1.0 / NC3)], axis=1)
        h = jnp.maximum(
            jnp.dot(r, w1_r[...], preferred_element_type=_f32) + b1_r[...], 0.0)
        y = jnp.sum(h * w2_r[...], axis=1, keepdims=True)
        o_r[...] = y + b2_r[...]

    return pl.pallas_call(
        body,
        grid=(1,),
        in_specs=[
            pl.BlockSpec((8, HID), lambda i: (0, 0)),
            pl.BlockSpec((8, HID), lambda i: (0, 0)),
            pl.BlockSpec((8, HID), lambda i: (0, 0)),
            pl.BlockSpec((8, HID), lambda i: (0, 0)),
            pl.BlockSpec((4 * HID, HID), lambda i: (0, 0)),
            pl.BlockSpec((1, HID), lambda i: (0, 0)),
            pl.BlockSpec((1, HID), lambda i: (0, 0)),
            pl.BlockSpec((1, HID), lambda i: (0, 0)),
        ],
        out_specs=pl.BlockSpec((1, HID), lambda i: (0, 0)),
        out_shape=jax.ShapeDtypeStruct((1, HID), _f32),
    )(csb, csk, cs2, cs3, w1, b1, w2t, b2p)


def _edges3(e_arr, e):
    src = e_arr[0].astype(jnp.int32).reshape(NWORK, e // NWORK // CH, CH)
    dst = e_arr[1].astype(jnp.int32).reshape(NWORK, e // NWORK // CH, CH)
    return src, dst


def _gin_tc(x, a_p, p, pfx, n, d):
    w1 = p[pfx + "_W1"]
    if w1.shape[0] < d:
        w1 = jnp.pad(w1, ((0, d - w1.shape[0]), (0, 0)))
    h1, st = _tc_a2_gen(x, a_p, w1, p[pfx + "_b1"].reshape(1, HID), n, d)
    return _tc_a3(h1, st, p[pfx + "_g1"].reshape(1, HID),
                  p[pfx + "_bt1"].reshape(1, HID), p[pfx + "_W2"],
                  p[pfx + "_b2"].reshape(1, HID), n)


def kernel(feats_A, pca_C2, pca_C3, params, edge_bond, edge_knn,
           g1_dst, edge_I2, g2_dst, edge_I3):
    p = params
    feats = feats_A.astype(_f32)

    rep4 = jnp.repeat(jnp.arange(HEADS), HID // HEADS)
    r16 = jnp.zeros((16, HID), _f32).at[rep4, jnp.arange(HID)].set(1.0)
    b16 = jnp.concatenate([jnp.zeros((4,), _f32),
                           jnp.full((12,), -1e30, _f32)]).reshape(1, 16)
    zs = jnp.zeros((NA, HID), _f32)
    zden = jnp.zeros((NA8, HID), _f32)

    hs = []
    colsums = []
    nchg = EB // NWORK // CHG
    for et, ei in (("bond", edge_bond), ("knn", edge_knn)):
        src3, dst3 = _edges3(ei, EB)
        src3g = src3.reshape(NWORK, nchg, CHG)
        dst3g = dst3.reshape(NWORK, nchg, CHG)
        al16 = jnp.zeros((HID, 16), _f32).at[jnp.arange(HID), rep4].set(
            p[et + "_al"].reshape(-1))
        ar16 = jnp.zeros((HID, 16), _f32).at[jnp.arange(HID), rep4].set(
            p[et + "_ar"].reshape(-1))
        g_tbl, er_tbl = _tc_zelter(feats, p[et + "_Wg"], al16, ar16, b16)
        d8_3 = dst3g // 8
        s_p, ex_e = _sc_gat(g_tbl, er_tbl, src3g, dst3g, zs)
        den_p = _sc_den(ex_e.reshape(NWORK, nchg, CHG, 16), dst3g, d8_3, zden)
        gat = _tc_gat_finish(s_p, den_p.reshape(NCORE, NA, 16), r16)
        az = _sc_seg_sum(gat.reshape(1, NA, HID), src3, dst3, zs,
                         NA, HID, EB).reshape(NCORE, NA, HID)
        ax = _sc_seg_sum(feats.reshape(1, NA, IN), src3, dst3, zs,
                         NA, IN, EB).reshape(NCORE, NA, IN)
        h1, st = _tc_a2_atoms(gat, feats, az, ax,
                              p[et + "_W1"][:HID], p[et + "_W1"][HID:],
                              p[et + "_b1"].reshape(1, HID))
        h_et, cs = _tc_a3(h1, st, p[et + "_g1"].reshape(1, HID),
                          p[et + "_bt1"].reshape(1, HID), p[et + "_W2"],
                          p[et + "_b2"].reshape(1, HID), NA)
        hs.append(h_et)
        colsums.append(cs)

    mb = _sc_seg_max(hs[0], g1_dst, NA, NC2).reshape(
        4, 8, NC2, 16).transpose(0, 2, 1, 3).reshape(4, NC2, 128)
    mk = _sc_seg_max(hs[1], g1_dst, NA, NC2).reshape(
        4, 8, NC2, 16).transpose(0, 2, 1, 3).reshape(4, NC2, 128)
    pca2 = pca_C2.astype(_f32).reshape(NC2, 16)
    h2cat = _tc_combine([mb, mk], pca2, NC2, 3 * HID)

    i2s, i2d = _edges3(edge_I2, EI2)
    z2 = jnp.zeros((NC2, HID), _f32)
    a_p = _sc_seg_sum(h2cat, i2s, i2d, z2, NC2, 3 * HID, EI2)
    h2g1, _ = _gin_tc(h2cat, a_p, p, "h2_0", NC2, 3 * HID)
    h2g1g = h2g1.reshape(1, NC2, HID)
    a_p = _sc_seg_sum(h2g1g, i2s, i2d, z2, NC2, HID, EI2)
    h2g2, cs2 = _gin_tc(h2g1g, a_p, p, "h2_1", NC2, HID)

    m3 = _sc_seg_max(h2g2, g2_dst, NC2, NC3).reshape(
        4, 8, NC3, 16).transpose(0, 2, 1, 3).reshape(4, NC3, 128)
    pca3 = pca_C3.astype(_f32).reshape(NC3, 16)
    h3cat = _tc_combine([m3], pca3, NC3, 2 * HID)

    i3s, i3d = _edges3(edge_I3, EI3)
    z3 = jnp.zeros((NC3, HID), _f32)
    a_p = _sc_seg_sum(h3cat, i3s, i3d, z3, NC3, 2 * HID, EI3)
    h3g1, _ = _gin_tc(h3cat, a_p, p, "h3_0", NC3, 2 * HID)
    h3g1g = h3g1.reshape(1, NC3, HID)
    a_p = _sc_seg_sum(h3g1g, i3s, i3d, z3, NC3, HID, EI3)
    h3g2, cs3 = _gin_tc(h3g1g, a_p, p, "h3_1", NC3, HID)

    b2p = jnp.zeros((1, HID), _f32).at[0, 0].set(p["out_b2"][0])
    y = _tc_readout(colsums[0], colsums[1], cs2, cs3,
                    p["out_W1"], p["out_b1"].reshape(1, HID),
                    p["out_W2"].reshape(1, HID), b2p)
    return y[0:1, 0:1]


# speedup vs baseline: 25.1198x; 1.3769x over previous
import functools

import jax
import jax.numpy as jnp
from jax import lax
from jax.experimental import pallas as pl
from jax.experimental.pallas import tpu as pltpu
from jax.experimental.pallas import tpu_sc as plsc

NA = 10000
IN = 128
HID = 128
HEADS = 4
NC2 = 2000
NC3 = 400
EB = 160000
EI2 = 32000
EI3 = 6400

NCORE = 2
NSUB = 16
NWORK = NCORE * NSUB
CH = 100

_f32 = jnp.float32


def _mesh():
    return plsc.VectorSubcoreMesh(
        core_axis_name="c", subcore_axis_name="s",
        num_cores=NCORE, num_subcores=NSUB)


def _row_split_copy(src, dst, s, n):
    rz = (n // NSUB // 8) * 8
    tail = n - rz * NSUB
    pltpu.sync_copy(src.at[pl.ds(s * rz, rz)], dst.at[pl.ds(s * rz, rz)])
    if tail:
        @pl.when(s == NSUB - 1)
        def _():
            pltpu.sync_copy(src.at[pl.ds(NSUB * rz, tail)],
                            dst.at[pl.ds(NSUB * rz, tail)])


def _sc_seg_sum(table, src3, dst3, zrows, n, d, e):
    K = d // 128
    ec = e // NWORK
    nch = ec // CH

    @functools.partial(
        pl.kernel,
        out_type=jax.ShapeDtypeStruct((NCORE, K, n, HID), _f32),
        mesh=_mesh(),
        scratch_types=[
            pltpu.VMEM((nch, CH), jnp.int32),
            pltpu.VMEM((nch, CH), jnp.int32),
            pltpu.VMEM((CH, HID), _f32),
            pltpu.VMEM((CH, HID), _f32),
            pltpu.VMEM_SHARED((K, n, HID), _f32),
            pltpu.SemaphoreType.DMA,
            pltpu.SemaphoreType.DMA,
        ],
    )
    def k(tbl, sr, ds_, zr, out, src_v, dst_v, rows0, rows1, acc, sem0, sem1):
        c = lax.axis_index("c")
        s = lax.axis_index("s")
        w = c * NSUB + s
        for kk in range(K):
            _row_split_copy(zr, acc.at[kk], s, n)
        pltpu.sync_copy(sr.at[w], src_v)
        pltpu.sync_copy(ds_.at[w], dst_v)
        plsc.subcore_barrier()

        if K == 1:
            def body(jj, carry):
                j0 = 2 * jj
                d0 = pltpu.async_copy(tbl.at[0].at[src_v.at[j0]], rows0, sem0)
                d1 = pltpu.async_copy(tbl.at[0].at[src_v.at[j0 + 1]], rows1,
                                      sem1)
                d0.wait()
                pltpu.sync_copy(rows0, acc.at[0].at[dst_v.at[j0]], add=True)
                d1.wait()
                pltpu.sync_copy(rows1, acc.at[0].at[dst_v.at[j0 + 1]],
                                add=True)
                return carry

            lax.fori_loop(0, nch // 2, body, 0)
        else:
            def body(j, carry):
                for kk in range(K):
                    pltpu.async_copy(tbl.at[kk].at[src_v.at[j]], rows0,
                                     sem0).wait()
                    pltpu.sync_copy(rows0, acc.at[kk].at[dst_v.at[j]],
                                    add=True)
                return carry

            lax.fori_loop(0, nch, body, 0)
        plsc.subcore_barrier()
        for kk in range(K):
            _row_split_copy(acc.at[kk], out.at[c, kk], s, n)

    return k(table, src3, dst3, zrows)


NA8 = NA // 8
CHG = 100


def _sc_gat(g_tbl, er_tbl, src3, dst3, zs):
    ec = EB // NWORK
    nch = ec // CHG
    H = CHG // 4
    tail = H % 16

    @functools.partial(
        pl.kernel,
        out_type=(jax.ShapeDtypeStruct((NCORE, NA, HID), _f32),
                  jax.ShapeDtypeStruct((NWORK, ec // H, H, 16), _f32)),
        mesh=_mesh(),
        scratch_types=[
            pltpu.VMEM((nch, CHG), jnp.int32),
            pltpu.VMEM((nch, CHG), jnp.int32),
            pltpu.VMEM((H, 2 * HID), _f32),
            pltpu.VMEM((H, 2 * HID), _f32),
            pltpu.VMEM((H, HID), _f32),
            pltpu.VMEM((H, HID), _f32),
            pltpu.VMEM((H, HID), _f32),
            pltpu.VMEM((H, 16), _f32),
            pltpu.VMEM_SHARED((NA, HID), _f32),
            pltpu.SemaphoreType.DMA,
            pltpu.SemaphoreType.DMA,
            pltpu.SemaphoreType.DMA,
            pltpu.SemaphoreType.DMA,
        ],
    )
    def k(gt, et, sr, ds_, zs_h, s_out, ex_out,
          src_v, dst_v, gb0, gb1, eb0, eb1, wb, exb, s_acc,
          sg0, se0, sg1, se1):
        c = lax.axis_index("c")
        s = lax.axis_index("s")
        w = c * NSUB + s
        _row_split_copy(zs_h, s_acc, s, NA)
        pltpu.sync_copy(sr.at[w], src_v)
        pltpu.sync_copy(ds_.at[w], dst_v)
        plsc.subcore_barrier()

        def do_edge(gb, i):
            elv = gb[i, pl.ds(HID, 16)]
            erv = eb0[i, pl.ds(0, 16)] if gb is gb0 else eb1[i, pl.ds(0, 16)]
            t = elv + erv
            ex = jnp.exp(jnp.maximum(t, 0.2 * t))
            exb[i] = ex
            for kk in range(8):
                exh = jnp.full((16,), ex[kk // 2], _f32)
                wb[i, pl.ds(kk * 16, 16)] = gb[i, pl.ds(kk * 16, 16)] * exh

        def process(j, q, gb):
            didx = dst_v.at[j, pl.ds(q * H, H)]

            def grp(b, cc):
                for kk in range(16):
                    do_edge(gb, b * 16 + kk)
                return cc

            lax.fori_loop(0, H // 16, grp, 0)
            if tail:
                for kk in range(16 - tail, 16):
                    do_edge(gb, H - 16 + kk)
            pltpu.sync_copy(wb, s_acc.at[didx], add=True)
            pltpu.sync_copy(exb, ex_out.at[w].at[4 * j + q])

        def pair(j, qa, qb):
            sa = src_v.at[j, pl.ds(qa * H, H)]
            da = dst_v.at[j, pl.ds(qa * H, H)]
            sb = src_v.at[j, pl.ds(qb * H, H)]
            db = dst_v.at[j, pl.ds(qb * H, H)]
            g0 = pltpu.async_copy(gt.at[sa], gb0, sg0)
            e0 = pltpu.async_copy(et.at[da], eb0, se0)
            g1 = pltpu.async_copy(gt.at[sb], gb1, sg1)
            e1 = pltpu.async_copy(et.at[db], eb1, se1)
            g0.wait()
            e0.wait()
            process(j, qa, gb0)
            g1.wait()
            e1.wait()
            process(j, qb, gb1)

        def chunk(j, carry):
            pair(j, 0, 1)
            pair(j, 2, 3)
            return carry

        lax.fori_loop(0, nch, chunk, 0)
        plsc.subcore_barrier()
        _row_split_copy(s_acc, s_out.at[c], s, NA)

    return k(g_tbl, er_tbl, src3, dst3, zs)


def _sc_den(ex_e, dst3, d8_3, zden):
    ec = EB // NWORK
    nch = ec // CHG
    tail = CHG % 16

    @functools.partial(
        pl.kernel,
        out_type=jax.ShapeDtypeStruct((NCORE, NA8, HID), _f32),
        mesh=_mesh(),
        scratch_types=[
            pltpu.VMEM((nch, CHG), jnp.int32),
            pltpu.VMEM((nch, CHG), jnp.int32),
            pltpu.VMEM((CHG, 16), _f32),
            pltpu.VMEM((CHG, HID), _f32),
            pltpu.VMEM_SHARED((NA8, HID), _f32),
            pltpu.SemaphoreType.DMA,
        ],
    )
    def k(exh_h, ds_, d8, zd_h, den_out,
          dst_v, d8_v, exb, db, d_acc, sem):
        c = lax.axis_index("c")
        s = lax.axis_index("s")
        w = c * NSUB + s
        _row_split_copy(zd_h, d_acc, s, NA8)
        pltpu.sync_copy(ds_.at[w], dst_v)
        pltpu.sync_copy(d8.at[w], d8_v)
        plsc.subcore_barrier()

        def do_edge(i, d, d8s):
            ex = exb[i]
            dm = d - d8s * 8
            zero = jnp.zeros((16,), _f32)
            for g in range(8):
                db[i, pl.ds(g * 16, 16)] = jnp.where(dm == g, ex, zero)

        def chunk(j, carry):
            pltpu.sync_copy(exh_h.at[w].at[j], exb)

            def grp(b, cc):
                dvv = dst_v[j, pl.ds(b * 16, 16)]
                dv8 = d8_v[j, pl.ds(b * 16, 16)]
                for kk in range(16):
                    do_edge(b * 16 + kk, dvv[kk], dv8[kk])
                return cc

            lax.fori_loop(0, CHG // 16, grp, 0)
            if tail:
                dvv = dst_v[j, pl.ds(CHG - 16, 16)]
                dv8 = d8_v[j, pl.ds(CHG - 16, 16)]
                for kk in range(16 - tail, 16):
                    do_edge(CHG - 16 + kk, dvv[kk], dv8[kk])
            pltpu.sync_copy(db, d_acc.at[d8_v.at[j]], add=True)
            return carry

        lax.fori_loop(0, nch, chunk, 0)
        plsc.subcore_barrier()
        _row_split_copy(d_acc, den_out.at[c], s, NA8)

    return k(ex_e, dst3, d8_3, zden)


def _sc_seg_max(table, seg, n, nseg):
    npad = ((n + 63) // 64) * 64
    n4 = npad // 4
    table_t = jnp.pad(table, ((0, npad - n), (0, 0))).reshape(
        npad, 8, 16).transpose(1, 0, 2).reshape(8, npad * 16)
    seg4 = jnp.pad(seg.astype(jnp.int32), (0, npad - n)).reshape(4, n4)

    @functools.partial(
        pl.kernel,
        out_type=jax.ShapeDtypeStruct((NCORE, 2, 8, nseg * 16), _f32),
        mesh=_mesh(),
        scratch_types=[
            pltpu.VMEM((nseg * 16,), _f32),
            pltpu.VMEM((n4 * 16,), _f32),
            pltpu.VMEM((n4,), jnp.int32),
            pltpu.SemaphoreType.DMA,
        ],
    )
    def k(tbl, sg, out, acc, rowb, seg_v, sem):
        c = lax.axis_index("c")
        s = lax.axis_index("s")
        g = s // 2
        q = s % 2
        r0 = (2 * c + q) * n4
        pltpu.sync_copy(sg.at[2 * c + q], seg_v)
        pltpu.sync_copy(tbl.at[g, pl.ds(r0 * 16, n4 * 16)], rowb)

        def zr(r, cc):
            acc[pl.ds(r * 16, 16)] = jnp.zeros((16,), _f32)
            return cc

        lax.fori_loop(0, nseg, zr, 0)

        def row16(cb, cc):
            sv = seg_v[pl.ds(cb * 16, 16)]
            for kk in range(16):
                off = sv[kk] * 16
                acc[pl.ds(off, 16)] = jnp.maximum(
                    acc[pl.ds(off, 16)], rowb[pl.ds((cb * 16 + kk) * 16, 16)])
            return cc

        lax.fori_loop(0, n4 // 16, row16, 0)
        pltpu.sync_copy(acc, out.at[c, q, g])

    return k(table_t, seg4)


def _tc_zelter(x, wg, al16, ar16, b16):
    br = 1000
    grid = NA // br

    def body(x_r, wg_r, al_r, ar_r, b_r, g_r, er_r):
        z = jnp.dot(x_r[...], wg_r[...], preferred_element_type=_f32)
        el16 = jnp.dot(z, al_r[...], preferred_element_type=_f32) + b_r[...]
        pad = jnp.zeros((br, HID - 16), _f32)
        g_r[...] = jnp.concatenate([z, el16, pad], axis=1)
        er16 = jnp.dot(z, ar_r[...], preferred_element_type=_f32)
        er_r[...] = jnp.concatenate([er16, pad], axis=1)

    return pl.pallas_call(
        body,
        grid=(grid,),
        in_specs=[
            pl.BlockSpec((br, IN), lambda i: (i, 0)),
            pl.BlockSpec((IN, HID), lambda i: (0, 0)),
            pl.BlockSpec((HID, 16), lambda i: (0, 0)),
            pl.BlockSpec((HID, 16), lambda i: (0, 0)),
            pl.BlockSpec((1, 16), lambda i: (0, 0)),
        ],
        out_specs=[
            pl.BlockSpec((br, 2 * HID), lambda i: (i, 0)),
            pl.BlockSpec((br, HID), lambda i: (i, 0)),
        ],
        out_shape=[
            jax.ShapeDtypeStruct((NA, 2 * HID), _f32),
            jax.ShapeDtypeStruct((NA, HID), _f32),
        ],
    )(x, wg, al16, ar16, b16)


def _tc_gat_finish(s_p, den16, r16):
    br = 1000
    grid = NA // br

    def body(s_r, d_r, r_r, o_r):
        ssum = s_r[0] + s_r[1]
        dsum = d_r[0] + d_r[1]
        dex = jnp.dot(dsum, r_r[...], preferred_element_type=_f32)
        o_r[...] = jnp.maximum(ssum / (dex + 1e-9), 0.0)

    return pl.pallas_call(
        body,
        grid=(grid,),
        in_specs=[
            pl.BlockSpec((2, br, HID), lambda i: (0, i, 0)),
            pl.BlockSpec((2, br, 16), lambda i: (0, i, 0)),
            pl.BlockSpec((16, HID), lambda i: (0, 0)),
        ],
        out_specs=pl.BlockSpec((br, HID), lambda i: (i, 0)),
        out_shape=jax.ShapeDtypeStruct((NA, HID), _f32),
    )(s_p, den16, r16)


def _stats_update(st_r, h, first):
    @pl.when(first)
    def _():
        st_r[...] = jnp.zeros_like(st_r)

    su = jnp.sum(h, axis=0, keepdims=True)
    sq = jnp.sum(h * h, axis=0, keepdims=True)
    st_r[...] += jnp.concatenate(
        [su, sq, jnp.zeros((6, HID), _f32)], axis=0)


def _tc_a2_atoms(gat, x, az, ax, w1a, w1b, b1):
    br = 1000
    grid = NA // br

    def body(g_r, x_r, az_r, ax_r, wa_r, wb_r, b_r, h_r, st_r):
        xa = g_r[...] + az_r[0] + az_r[1]
        xb = x_r[...] + ax_r[0] + ax_r[1]
        h = (jnp.dot(xa, wa_r[...], preferred_element_type=_f32)
             + jnp.dot(xb, wb_r[...], preferred_element_type=_f32) + b_r[...])
        h_r[...] = h
        _stats_update(st_r, h, pl.program_id(0) == 0)

    return pl.pallas_call(
        body,
        grid=(grid,),
        in_specs=[
            pl.BlockSpec((br, HID), lambda i: (i, 0)),
            pl.BlockSpec((br, IN), lambda i: (i, 0)),
            pl.BlockSpec((2, br, HID), lambda i: (0, i, 0)),
            pl.BlockSpec((2, br, IN), lambda i: (0, i, 0)),
            pl.BlockSpec((HID, HID), lambda i: (0, 0)),
            pl.BlockSpec((IN, HID), lambda i: (0, 0)),
            pl.BlockSpec((1, HID), lambda i: (0, 0)),
        ],
        out_specs=[
            pl.BlockSpec((br, HID), lambda i: (i, 0)),
            pl.BlockSpec((8, HID), lambda i: (0, 0)),
        ],
        out_shape=[
            jax.ShapeDtypeStruct((NA, HID), _f32),
            jax.ShapeDtypeStruct((8, HID), _f32),
        ],
    )(gat, x, az, ax, w1a, w1b, b1)


def _tc_a2_gen(x, a_p, w1, b1, n, d):
    br = min(n, 1000)
    grid = n // br
    K = d // 128

    def body(x_r, a_r, w_r, b_r, h_r, st_r):
        xin = jnp.concatenate(
            [x_r[kk] + a_r[0, kk] + a_r[1, kk] for kk in range(K)], axis=1)
        h = jnp.dot(xin, w_r[...], preferred_element_type=_f32) + b_r[...]
        h_r[...] = h
        _stats_update(st_r, h, pl.program_id(0) == 0)

    return pl.pallas_call(
        body,
        grid=(grid,),
        in_specs=[
            pl.BlockSpec((K, br, 128), lambda i: (0, i, 0)),
            pl.BlockSpec((2, K, br, 128), lambda i: (0, 0, i, 0)),
            pl.BlockSpec((d, HID), lambda i: (0, 0)),
            pl.BlockSpec((1, HID), lambda i: (0, 0)),
        ],
        out_specs=[
            pl.BlockSpec((br, HID), lambda i: (i, 0)),
            pl.BlockSpec((8, HID), lambda i: (0, 0)),
        ],
        out_shape=[
            jax.ShapeDtypeStruct((n, HID), _f32),
            jax.ShapeDtypeStruct((8, HID), _f32),
        ],
    )(x, a_p, w1, b1)


def _tc_a3(h1, st, g, bt, w2, b2, n):
    br = min(n, 1000)
    grid = n // br
    inv_n = 1.0 / n

    def body(h_r, st_r, g_r, bt_r, w2_r, b2_r, o_r, cs_r):
        stv = st_r[...]
        mu = stv[0:1] * inv_n
        var = stv[1:2] * inv_n - mu * mu
        sc = g_r[...] * lax.rsqrt(var + 1e-5)
        hn = (h_r[...] - mu) * sc + bt_r[...]
        o = jnp.dot(jnp.maximum(hn, 0.0), w2_r[...],
                    preferred_element_type=_f32) + b2_r[...]
        o = jnp.maximum(o, 0.0)
        o_r[...] = o

        @pl.when(pl.program_id(0) == 0)
        def _():
            cs_r[...] = jnp.zeros_like(cs_r)

        cs_r[...] += jnp.concatenate(
            [jnp.sum(o, axis=0, keepdims=True), jnp.zeros((7, HID), _f32)], axis=0)

    return pl.pallas_call(
        body,
        grid=(grid,),
        in_specs=[
            pl.BlockSpec((br, HID), lambda i: (i, 0)),
            pl.BlockSpec((8, HID), lambda i: (0, 0)),
            pl.BlockSpec((1, HID), lambda i: (0, 0)),
            pl.BlockSpec((1, HID), lambda i: (0, 0)),
            pl.BlockSpec((HID, HID), lambda i: (0, 0)),
            pl.BlockSpec((1, HID), lambda i: (0, 0)),
        ],
        out_specs=[
            pl.BlockSpec((br, HID), lambda i: (i, 0)),
            pl.BlockSpec((8, HID), lambda i: (0, 0)),
        ],
        out_shape=[
            jax.ShapeDtypeStruct((n, HID), _f32),
            jax.ShapeDtypeStruct((8, HID), _f32),
        ],
    )(h1, st, g, bt, w2, b2)


def _tc_combine(pairs, pca, n, dout):
    br = min(n, 1000)
    grid = n // br
    npair = len(pairs)
    K = dout // 128
    dpad = dout - 128 * npair - 16

    def body(*refs):
        prs = refs[:npair]
        p_r = refs[npair]
        o_r = refs[npair + 1]
        for kk, m in enumerate(prs):
            o_r[kk] = jnp.maximum(jnp.maximum(m[0], m[1]),
                                  jnp.maximum(m[2], m[3]))
        o_r[npair] = jnp.concatenate(
            [p_r[...], jnp.zeros((br, dpad), _f32)], axis=1)

    in_specs = [pl.BlockSpec((4, br, 128), lambda i: (0, i, 0))
                for _ in pairs]
    in_specs.append(pl.BlockSpec((br, 16), lambda i: (i, 0)))
    return pl.pallas_call(
        body,
        grid=(grid,),
        in_specs=in_specs,
        out_specs=pl.BlockSpec((K, br, 128), lambda i: (0, i, 0)),
        out_shape=jax.ShapeDtypeStruct((K, n, 128), _f32),
    )(*pairs, pca)


def _tc_readout(csb, csk, cs2, cs3, w1, b1, w2t, b2p):
    def body(cb, ck, c2, c3, w1_r, b1_r, w2_r, b2_r, o_r):
        r = jnp.concatenate(
            [cb[0:1] * (1.0 / NA), ck[0:1] * (1.0 / NA),
             c2[0:1] * (1.0 / NC2), c3[0:1] * (1.0 / NC3)], axis=1)
        h = jnp.maximum(
            jnp.dot(r, w1_r[...], preferred_element_type=_f32) + b1_r[...], 0.0)
        y = jnp.sum(h * w2_r[...], axis=1, keepdims=True)
        o_r[...] = y + b2_r[...]

    return pl.pallas_call(
        body,
        grid=(1,),
        in_specs=[
            pl.BlockSpec((8, HID), lambda i: (0, 0)),
            pl.BlockSpec((8, HID), lambda i: (0, 0)),
            pl.BlockSpec((8, HID), lambda i: (0, 0)),
            pl.BlockSpec((8, HID), lambda i: (0, 0)),
            pl.BlockSpec((4 * HID, HID), lambda i: (0, 0)),
            pl.BlockSpec((1, HID), lambda i: (0, 0)),
            pl.BlockSpec((1, HID), lambda i: (0, 0)),
            pl.BlockSpec((1, HID), lambda i: (0, 0)),
        ],
        out_specs=pl.BlockSpec((1, HID), lambda i: (0, 0)),
        out_shape=jax.ShapeDtypeStruct((1, HID), _f32),
    )(csb, csk, cs2, cs3, w1, b1, w2t, b2p)


def _edges3(e_arr, e):
    src = e_arr[0].astype(jnp.int32).reshape(NWORK, e // NWORK // CH, CH)
    dst = e_arr[1].astype(jnp.int32).reshape(NWORK, e // NWORK // CH, CH)
    return src, dst


def _gin_tc(x, a_p, p, pfx, n, d):
    w1 = p[pfx + "_W1"]
    if w1.shape[0] < d:
        w1 = jnp.pad(w1, ((0, d - w1.shape[0]), (0, 0)))
    h1, st = _tc_a2_gen(x, a_p, w1, p[pfx + "_b1"].reshape(1, HID), n, d)
    return _tc_a3(h1, st, p[pfx + "_g1"].reshape(1, HID),
                  p[pfx + "_bt1"].reshape(1, HID), p[pfx + "_W2"],
                  p[pfx + "_b2"].reshape(1, HID), n)


def kernel(feats_A, pca_C2, pca_C3, params, edge_bond, edge_knn,
           g1_dst, edge_I2, g2_dst, edge_I3):
    p = params
    feats = feats_A.astype(_f32)

    rep4 = jnp.repeat(jnp.arange(HEADS), HID // HEADS)
    r16 = jnp.zeros((16, HID), _f32).at[rep4, jnp.arange(HID)].set(1.0)
    b16 = jnp.concatenate([jnp.zeros((4,), _f32),
                           jnp.full((12,), -1e30, _f32)]).reshape(1, 16)
    zs = jnp.zeros((NA, HID), _f32)
    zden = jnp.zeros((NA8, HID), _f32)

    hs = []
    colsums = []
    nchg = EB // NWORK // CHG
    for et, ei in (("bond", edge_bond), ("knn", edge_knn)):
        src3, dst3 = _edges3(ei, EB)
        src3g = src3.reshape(NWORK, nchg, CHG)
        dst3g = dst3.reshape(NWORK, nchg, CHG)
        al16 = jnp.zeros((HID, 16), _f32).at[jnp.arange(HID), rep4].set(
            p[et + "_al"].reshape(-1))
        ar16 = jnp.zeros((HID, 16), _f32).at[jnp.arange(HID), rep4].set(
            p[et + "_ar"].reshape(-1))
        g_tbl, er_tbl = _tc_zelter(feats, p[et + "_Wg"], al16, ar16, b16)
        d8_3 = dst3g // 8
        s_p, ex_e = _sc_gat(g_tbl, er_tbl, src3g, dst3g, zs)
        den_p = _sc_den(ex_e.reshape(NWORK, nchg, CHG, 16), dst3g, d8_3, zden)
        gat = _tc_gat_finish(s_p, den_p.reshape(NCORE, NA, 16), r16)
        az = _sc_seg_sum(gat.reshape(1, NA, HID), src3, dst3, zs,
                         NA, HID, EB).reshape(NCORE, NA, HID)
        ax = _sc_seg_sum(feats.reshape(1, NA, IN), src3, dst3, zs,
                         NA, IN, EB).reshape(NCORE, NA, IN)
        h1, st = _tc_a2_atoms(gat, feats, az, ax,
                              p[et + "_W1"][:HID], p[et + "_W1"][HID:],
                              p[et + "_b1"].reshape(1, HID))
        h_et, cs = _tc_a3(h1, st, p[et + "_g1"].reshape(1, HID),
                          p[et + "_bt1"].reshape(1, HID), p[et + "_W2"],
                          p[et + "_b2"].reshape(1, HID), NA)
        hs.append(h_et)
        colsums.append(cs)

    mb = _sc_seg_max(hs[0], g1_dst, NA, NC2).reshape(
        4, 8, NC2, 16).transpose(0, 2, 1, 3).reshape(4, NC2, 128)
    mk = _sc_seg_max(hs[1], g1_dst, NA, NC2).reshape(
        4, 8, NC2, 16).transpose(0, 2, 1, 3).reshape(4, NC2, 128)
    pca2 = pca_C2.astype(_f32).reshape(NC2, 16)
    h2cat = _tc_combine([mb, mk], pca2, NC2, 3 * HID)

    i2s, i2d = _edges3(edge_I2, EI2)
    z2 = jnp.zeros((NC2, HID), _f32)
    a_p = _sc_seg_sum(h2cat, i2s, i2d, z2, NC2, 3 * HID, EI2)
    h2g1, _ = _gin_tc(h2cat, a_p, p, "h2_0", NC2, 3 * HID)
    h2g1g = h2g1.reshape(1, NC2, HID)
    a_p = _sc_seg_sum(h2g1g, i2s, i2d, z2, NC2, HID, EI2)
    h2g2, cs2 = _gin_tc(h2g1g, a_p, p, "h2_1", NC2, HID)

    m3 = _sc_seg_max(h2g2, g2_dst, NC2, NC3).reshape(
        4, 8, NC3, 16).transpose(0, 2, 1, 3).reshape(4, NC3, 128)
    pca3 = pca_C3.astype(_f32).reshape(NC3, 16)
    h3cat = _tc_combine([m3], pca3, NC3, 2 * HID)

    i3s, i3d = _edges3(edge_I3, EI3)
    z3 = jnp.zeros((NC3, HID), _f32)
    a_p = _sc_seg_sum(h3cat, i3s, i3d, z3, NC3, 2 * HID, EI3)
    h3g1, _ = _gin_tc(h3cat, a_p, p, "h3_0", NC3, 2 * HID)
    h3g1g = h3g1.reshape(1, NC3, HID)
    a_p = _sc_seg_sum(h3g1g, i3s, i3d, z3, NC3, HID, EI3)
    h3g2, cs3 = _gin_tc(h3g1g, a_p, p, "h3_1", NC3, HID)

    b2p = jnp.zeros((1, HID), _f32).at[0, 0].set(p["out_b2"][0])
    y = _tc_readout(colsums[0], colsums[1], cs2, cs3,
                    p["out_W1"], p["out_b1"].reshape(1, HID),
                    p["out_W2"].reshape(1, HID), b2p)
    return y[0:1, 0:1]


# speedup vs baseline: 25.5895x; 1.0187x over previous
import functools

import jax
import jax.numpy as jnp
from jax import lax
from jax.experimental import pallas as pl
from jax.experimental.pallas import tpu as pltpu
from jax.experimental.pallas import tpu_sc as plsc

NA = 10000
IN = 128
HID = 128
HEADS = 4
NC2 = 2000
NC3 = 400
EB = 160000
EI2 = 32000
EI3 = 6400

NCORE = 2
NSUB = 16
NWORK = NCORE * NSUB
CH = 100

_f32 = jnp.float32


def _mesh():
    return plsc.VectorSubcoreMesh(
        core_axis_name="c", subcore_axis_name="s",
        num_cores=NCORE, num_subcores=NSUB)


def _row_split_copy(src, dst, s, n):
    rz = (n // NSUB // 8) * 8
    tail = n - rz * NSUB
    pltpu.sync_copy(src.at[pl.ds(s * rz, rz)], dst.at[pl.ds(s * rz, rz)])
    if tail:
        @pl.when(s == NSUB - 1)
        def _():
            pltpu.sync_copy(src.at[pl.ds(NSUB * rz, tail)],
                            dst.at[pl.ds(NSUB * rz, tail)])


def _sc_seg_sum(table, src3, dst3, zrows, n, d, e):
    K = d // 128
    ec = e // NWORK
    nch = ec // CH

    @functools.partial(
        pl.kernel,
        out_type=jax.ShapeDtypeStruct((NCORE, K, n, HID), _f32),
        mesh=_mesh(),
        scratch_types=[
            pltpu.VMEM((nch, CH), jnp.int32),
            pltpu.VMEM((nch, CH), jnp.int32),
            pltpu.VMEM((CH, HID), _f32),
            pltpu.VMEM((CH, HID), _f32),
            pltpu.VMEM_SHARED((K, n, HID), _f32),
            pltpu.SemaphoreType.DMA,
            pltpu.SemaphoreType.DMA,
        ],
    )
    def k(tbl, sr, ds_, zr, out, src_v, dst_v, rows0, rows1, acc, sem0, sem1):
        c = lax.axis_index("c")
        s = lax.axis_index("s")
        w = c * NSUB + s
        for kk in range(K):
            _row_split_copy(zr, acc.at[kk], s, n)
        pltpu.sync_copy(sr.at[w], src_v)
        pltpu.sync_copy(ds_.at[w], dst_v)
        plsc.subcore_barrier()

        def body(mm, carry):
            m0 = 2 * mm
            m1 = m0 + 1
            j0 = m0 // K
            k0 = m0 - j0 * K
            j1 = m1 // K
            k1 = m1 - j1 * K
            d0 = pltpu.async_copy(tbl.at[k0].at[src_v.at[j0]], rows0, sem0)
            d1 = pltpu.async_copy(tbl.at[k1].at[src_v.at[j1]], rows1, sem1)
            d0.wait()
            pltpu.sync_copy(rows0, acc.at[k0].at[dst_v.at[j0]], add=True)
            d1.wait()
            pltpu.sync_copy(rows1, acc.at[k1].at[dst_v.at[j1]], add=True)
            return carry

        lax.fori_loop(0, nch * K // 2, body, 0)
        plsc.subcore_barrier()
        for kk in range(K):
            _row_split_copy(acc.at[kk], out.at[c, kk], s, n)

    return k(table, src3, dst3, zrows)


NA8 = NA // 8
CHG = 100


def _sc_gat(g_tbl, er_tbl, src3, dst3, zs):
    ec = EB // NWORK
    nch = ec // CHG
    H = CHG // 4
    tail = H % 16

    @functools.partial(
        pl.kernel,
        out_type=(jax.ShapeDtypeStruct((NCORE, NA, HID), _f32),
                  jax.ShapeDtypeStruct((NWORK, ec // H, H, 16), _f32)),
        mesh=_mesh(),
        scratch_types=[
            pltpu.VMEM((nch, CHG), jnp.int32),
            pltpu.VMEM((nch, CHG), jnp.int32),
            pltpu.VMEM((H, 2 * HID), _f32),
            pltpu.VMEM((H, 2 * HID), _f32),
            pltpu.VMEM((H, HID), _f32),
            pltpu.VMEM((H, HID), _f32),
            pltpu.VMEM((H, HID), _f32),
            pltpu.VMEM((H, 16), _f32),
            pltpu.VMEM_SHARED((NA, HID), _f32),
            pltpu.SemaphoreType.DMA,
            pltpu.SemaphoreType.DMA,
            pltpu.SemaphoreType.DMA,
            pltpu.SemaphoreType.DMA,
        ],
    )
    def k(gt, et, sr, ds_, zs_h, s_out, ex_out,
          src_v, dst_v, gb0, gb1, eb0, eb1, wb, exb, s_acc,
          sg0, se0, sg1, se1):
        c = lax.axis_index("c")
        s = lax.axis_index("s")
        w = c * NSUB + s
        _row_split_copy(zs_h, s_acc, s, NA)
        pltpu.sync_copy(sr.at[w], src_v)
        pltpu.sync_copy(ds_.at[w], dst_v)
        plsc.subcore_barrier()

        def do_edge(gb, i):
            elv = gb[i, pl.ds(HID, 16)]
            erv = eb0[i, pl.ds(0, 16)] if gb is gb0 else eb1[i, pl.ds(0, 16)]
            t = elv + erv
            ex = jnp.exp(jnp.maximum(t, 0.2 * t))
            exb[i] = ex
            for kk in range(8):
                exh = jnp.full((16,), ex[kk // 2], _f32)
                wb[i, pl.ds(kk * 16, 16)] = gb[i, pl.ds(kk * 16, 16)] * exh

        def process(j, q, gb):
            didx = dst_v.at[j, pl.ds(q * H, H)]

            def grp(b, cc):
                for kk in range(16):
                    do_edge(gb, b * 16 + kk)
                return cc

            lax.fori_loop(0, H // 16, grp, 0)
            if tail:
                for kk in range(16 - tail, 16):
                    do_edge(gb, H - 16 + kk)
            pltpu.sync_copy(wb, s_acc.at[didx], add=True)
            pltpu.sync_copy(exb, ex_out.at[w].at[4 * j + q])

        def pair(j, qa, qb):
            sa = src_v.at[j, pl.ds(qa * H, H)]
            da = dst_v.at[j, pl.ds(qa * H, H)]
            sb = src_v.at[j, pl.ds(qb * H, H)]
            db = dst_v.at[j, pl.ds(qb * H, H)]
            g0 = pltpu.async_copy(gt.at[sa], gb0, sg0)
            e0 = pltpu.async_copy(et.at[da], eb0, se0)
            g1 = pltpu.async_copy(gt.at[sb], gb1, sg1)
            e1 = pltpu.async_copy(et.at[db], eb1, se1)
            g0.wait()
            e0.wait()
            process(j, qa, gb0)
            g1.wait()
            e1.wait()
            process(j, qb, gb1)

        def chunk(j, carry):
            pair(j, 0, 1)
            pair(j, 2, 3)
            return carry

        lax.fori_loop(0, nch, chunk, 0)
        plsc.subcore_barrier()
        _row_split_copy(s_acc, s_out.at[c], s, NA)

    return k(g_tbl, er_tbl, src3, dst3, zs)


def _sc_den(ex_e, dst3, d8_3, zden):
    ec = EB // NWORK
    nch = ec // CHG
    tail = CHG % 16

    @functools.partial(
        pl.kernel,
        out_type=jax.ShapeDtypeStruct((NCORE, NA8, HID), _f32),
        mesh=_mesh(),
        scratch_types=[
            pltpu.VMEM((nch, CHG), jnp.int32),
            pltpu.VMEM((nch, CHG), jnp.int32),
            pltpu.VMEM((CHG, 16), _f32),
            pltpu.VMEM((CHG, 16), _f32),
            pltpu.VMEM((CHG, HID), _f32),
            pltpu.VMEM_SHARED((NA8, HID), _f32),
            pltpu.SemaphoreType.DMA,
            pltpu.SemaphoreType.DMA,
        ],
    )
    def k(exh_h, ds_, d8, zd_h, den_out,
          dst_v, d8_v, exb0, exb1, db, d_acc, sem0, sem1):
        c = lax.axis_index("c")
        s = lax.axis_index("s")
        w = c * NSUB + s
        _row_split_copy(zd_h, d_acc, s, NA8)
        pltpu.sync_copy(ds_.at[w], dst_v)
        pltpu.sync_copy(d8.at[w], d8_v)
        plsc.subcore_barrier()

        def do_edge(exb, i, d, d8s):
            ex = exb[i]
            dm = d - d8s * 8
            zero = jnp.zeros((16,), _f32)
            for g in range(8):
                db[i, pl.ds(g * 16, 16)] = jnp.where(dm == g, ex, zero)

        def process(j, exb):
            def grp(b, cc):
                dvv = dst_v[j, pl.ds(b * 16, 16)]
                dv8 = d8_v[j, pl.ds(b * 16, 16)]
                for kk in range(16):
                    do_edge(exb, b * 16 + kk, dvv[kk], dv8[kk])
                return cc

            lax.fori_loop(0, CHG // 16, grp, 0)
            if tail:
                dvv = dst_v[j, pl.ds(CHG - 16, 16)]
                dv8 = d8_v[j, pl.ds(CHG - 16, 16)]
                for kk in range(16 - tail, 16):
                    do_edge(exb, CHG - 16 + kk, dvv[kk], dv8[kk])
            pltpu.sync_copy(db, d_acc.at[d8_v.at[j]], add=True)

        def chunk(jj, carry):
            j0 = 2 * jj
            d0 = pltpu.async_copy(exh_h.at[w].at[j0], exb0, sem0)
            d1 = pltpu.async_copy(exh_h.at[w].at[j0 + 1], exb1, sem1)
            d0.wait()
            process(j0, exb0)
            d1.wait()
            process(j0 + 1, exb1)
            return carry

        lax.fori_loop(0, nch // 2, chunk, 0)
        plsc.subcore_barrier()
        _row_split_copy(d_acc, den_out.at[c], s, NA8)

    return k(ex_e, dst3, d8_3, zden)


def _sc_seg_max(table, seg, n, nseg):
    npad = ((n + 63) // 64) * 64
    n4 = npad // 4
    table_t = jnp.pad(table, ((0, npad - n), (0, 0))).reshape(
        npad, 8, 16).transpose(1, 0, 2).reshape(8, npad * 16)
    seg4 = jnp.pad(seg.astype(jnp.int32), (0, npad - n)).reshape(4, n4)

    @functools.partial(
        pl.kernel,
        out_type=jax.ShapeDtypeStruct((NCORE, 2, 8, nseg * 16), _f32),
        mesh=_mesh(),
        scratch_types=[
            pltpu.VMEM((nseg * 16,), _f32),
            pltpu.VMEM((n4 * 16,), _f32),
            pltpu.VMEM((n4,), jnp.int32),
            pltpu.SemaphoreType.DMA,
        ],
    )
    def k(tbl, sg, out, acc, rowb, seg_v, sem):
        c = lax.axis_index("c")
        s = lax.axis_index("s")
        g = s // 2
        q = s % 2
        r0 = (2 * c + q) * n4
        pltpu.sync_copy(sg.at[2 * c + q], seg_v)
        pltpu.sync_copy(tbl.at[g, pl.ds(r0 * 16, n4 * 16)], rowb)

        def zr(r, cc):
            acc[pl.ds(r * 16, 16)] = jnp.zeros((16,), _f32)
            return cc

        lax.fori_loop(0, nseg, zr, 0)

        def row16(cb, cc):
            sv = seg_v[pl.ds(cb * 16, 16)]
            for kk in range(16):
                off = sv[kk] * 16
                acc[pl.ds(off, 16)] = jnp.maximum(
                    acc[pl.ds(off, 16)], rowb[pl.ds((cb * 16 + kk) * 16, 16)])
            return cc

        lax.fori_loop(0, n4 // 16, row16, 0)
        pltpu.sync_copy(acc, out.at[c, q, g])

    return k(table_t, seg4)


def _tc_zelter(x, wg, al16, ar16, b16):
    br = 1000
    grid = NA // br

    def body(x_r, wg_r, al_r, ar_r, b_r, g_r, er_r):
        z = jnp.dot(x_r[...], wg_r[...], preferred_element_type=_f32)
        el16 = jnp.dot(z, al_r[...], preferred_element_type=_f32) + b_r[...]
        pad = jnp.zeros((br, HID - 16), _f32)
        g_r[...] = jnp.concatenate([z, el16, pad], axis=1)
        er16 = jnp.dot(z, ar_r[...], preferred_element_type=_f32)
        er_r[...] = jnp.concatenate([er16, pad], axis=1)

    return pl.pallas_call(
        body,
        grid=(grid,),
        in_specs=[
            pl.BlockSpec((br, IN), lambda i: (i, 0)),
            pl.BlockSpec((IN, HID), lambda i: (0, 0)),
            pl.BlockSpec((HID, 16), lambda i: (0, 0)),
            pl.BlockSpec((HID, 16), lambda i: (0, 0)),
            pl.BlockSpec((1, 16), lambda i: (0, 0)),
        ],
        out_specs=[
            pl.BlockSpec((br, 2 * HID), lambda i: (i, 0)),
            pl.BlockSpec((br, HID), lambda i: (i, 0)),
        ],
        out_shape=[
            jax.ShapeDtypeStruct((NA, 2 * HID), _f32),
            jax.ShapeDtypeStruct((NA, HID), _f32),
        ],
    )(x, wg, al16, ar16, b16)


def _tc_gat_finish(s_p, den16, r16):
    br = 1000
    grid = NA // br

    def body(s_r, d_r, r_r, o_r):
        ssum = s_r[0] + s_r[1]
        dsum = d_r[0] + d_r[1]
        dex = jnp.dot(dsum, r_r[...], preferred_element_type=_f32)
        o_r[...] = jnp.maximum(ssum / (dex + 1e-9), 0.0)

    return pl.pallas_call(
        body,
        grid=(grid,),
        in_specs=[
            pl.BlockSpec((2, br, HID), lambda i: (0, i, 0)),
            pl.BlockSpec((2, br, 16), lambda i: (0, i, 0)),
            pl.BlockSpec((16, HID), lambda i: (0, 0)),
        ],
        out_specs=pl.BlockSpec((br, HID), lambda i: (i, 0)),
        out_shape=jax.ShapeDtypeStruct((NA, HID), _f32),
    )(s_p, den16, r16)


def _stats_update(st_r, h, first):
    @pl.when(first)
    def _():
        st_r[...] = jnp.zeros_like(st_r)

    su = jnp.sum(h, axis=0, keepdims=True)
    sq = jnp.sum(h * h, axis=0, keepdims=True)
    st_r[...] += jnp.concatenate(
        [su, sq, jnp.zeros((6, HID), _f32)], axis=0)


def _tc_a2_atoms(gat, x, azax, w1a, w1b, b1):
    br = 1000
    grid = NA // br

    def body(g_r, x_r, a_r, wa_r, wb_r, b_r, h_r, st_r):
        xa = g_r[...] + a_r[0, 0] + a_r[1, 0]
        xb = x_r[...] + a_r[0, 1] + a_r[1, 1]
        h = (jnp.dot(xa, wa_r[...], preferred_element_type=_f32)
             + jnp.dot(xb, wb_r[...], preferred_element_type=_f32) + b_r[...])
        h_r[...] = h
        _stats_update(st_r, h, pl.program_id(0) == 0)

    return pl.pallas_call(
        body,
        grid=(grid,),
        in_specs=[
            pl.BlockSpec((br, HID), lambda i: (i, 0)),
            pl.BlockSpec((br, IN), lambda i: (i, 0)),
            pl.BlockSpec((2, 2, br, HID), lambda i: (0, 0, i, 0)),
            pl.BlockSpec((HID, HID), lambda i: (0, 0)),
            pl.BlockSpec((IN, HID), lambda i: (0, 0)),
            pl.BlockSpec((1, HID), lambda i: (0, 0)),
        ],
        out_specs=[
            pl.BlockSpec((br, HID), lambda i: (i, 0)),
            pl.BlockSpec((8, HID), lambda i: (0, 0)),
        ],
        out_shape=[
            jax.ShapeDtypeStruct((NA, HID), _f32),
            jax.ShapeDtypeStruct((8, HID), _f32),
        ],
    )(gat, x, azax, w1a, w1b, b1)


def _tc_a2_gen(x, a_p, w1, b1, n, d):
    br = min(n, 1000)
    grid = n // br
    K = d // 128

    def body(x_r, a_r, w_r, b_r, h_r, st_r):
        xin = jnp.concatenate(
            [x_r[kk] + a_r[0, kk] + a_r[1, kk] for kk in range(K)], axis=1)
        h = jnp.dot(xin, w_r[...], preferred_element_type=_f32) + b_r[...]
        h_r[...] = h
        _stats_update(st_r, h, pl.program_id(0) == 0)

    return pl.pallas_call(
        body,
        grid=(grid,),
        in_specs=[
            pl.BlockSpec((K, br, 128), lambda i: (0, i, 0)),
            pl.BlockSpec((2, K, br, 128), lambda i: (0, 0, i, 0)),
            pl.BlockSpec((d, HID), lambda i: (0, 0)),
            pl.BlockSpec((1, HID), lambda i: (0, 0)),
        ],
        out_specs=[
            pl.BlockSpec((br, HID), lambda i: (i, 0)),
            pl.BlockSpec((8, HID), lambda i: (0, 0)),
        ],
        out_shape=[
            jax.ShapeDtypeStruct((n, HID), _f32),
            jax.ShapeDtypeStruct((8, HID), _f32),
        ],
    )(x, a_p, w1, b1)


def _tc_a3(h1, st, g, bt, w2, b2, n):
    br = min(n, 1000)
    grid = n // br
    inv_n = 1.0 / n

    def body(h_r, st_r, g_r, bt_r, w2_r, b2_r, o_r, cs_r):
        stv = st_r[...]
        mu = stv[0:1] * inv_n
        var = stv[1:2] * inv_n - mu * mu
        sc = g_r[...] * lax.rsqrt(var + 1e-5)
        hn = (h_r[...] - mu) * sc + bt_r[...]
        o = jnp.dot(jnp.maximum(hn, 0.0), w2_r[...],
                    preferred_element_type=_f32) + b2_r[...]
        o = jnp.maximum(o, 0.0)
        o_r[...] = o

        @pl.when(pl.program_id(0) == 0)
        def _():
            cs_r[...] = jnp.zeros_like(cs_r)

        cs_r[...] += jnp.concatenate(
            [jnp.sum(o, axis=0, keepdims=True), jnp.zeros((7, HID), _f32)], axis=0)

    return pl.pallas_call(
        body,
        grid=(grid,),
        in_specs=[
            pl.BlockSpec((br, HID), lambda i: (i, 0)),
            pl.BlockSpec((8, HID), lambda i: (0, 0)),
            pl.BlockSpec((1, HID), lambda i: (0, 0)),
            pl.BlockSpec((1, HID), lambda i: (0, 0)),
            pl.BlockSpec((HID, HID), lambda i: (0, 0)),
            pl.BlockSpec((1, HID), lambda i: (0, 0)),
        ],
        out_specs=[
            pl.BlockSpec((br, HID), lambda i: (i, 0)),
            pl.BlockSpec((8, HID), lambda i: (0, 0)),
        ],
        out_shape=[
            jax.ShapeDtypeStruct((n, HID), _f32),
            jax.ShapeDtypeStruct((8, HID), _f32),
        ],
    )(h1, st, g, bt, w2, b2)


def _tc_combine(pairs, pca, n, dout):
    br = min(n, 1000)
    grid = n // br
    npair = len(pairs)
    K = dout // 128
    dpad = dout - 128 * npair - 16

    def body(*refs):
        prs = refs[:npair]
        p_r = refs[npair]
        o_r = refs[npair + 1]
        for kk, m in enumerate(prs):
            o_r[kk] = jnp.maximum(jnp.maximum(m[0], m[1]),
                                  jnp.maximum(m[2], m[3]))
        o_r[npair] = jnp.concatenate(
            [p_r[...], jnp.zeros((br, dpad), _f32)], axis=1)

    in_specs = [pl.BlockSpec((4, br, 128), lambda i: (0, i, 0))
                for _ in pairs]
    in_specs.append(pl.BlockSpec((br, 16), lambda i: (i, 0)))
    return pl.pallas_call(
        body,
        grid=(grid,),
        in_specs=in_specs,
        out_specs=pl.BlockSpec((K, br, 128), lambda i: (0, i, 0)),
        out_shape=jax.ShapeDtypeStruct((K, n, 128), _f32),
    )(*pairs, pca)


def _tc_readout(csb, csk, cs2, cs3, w1, b1, w2t, b2p):
    def body(cb, ck, c2, c3, w1_r, b1_r, w2_r, b2_r, o_r):
        r = jnp.concatenate(
            [cb[0:1] * (1.0 / NA), ck[0:1] * (1.0 / NA),
             c2[0:1] * (1.0 / NC2), c3[0:1] * (1.0 / NC3)], axis=1)
        h = jnp.maximum(
            jnp.dot(r, w1_r[...], preferred_element_type=_f32) + b1_r[...], 0.0)
        y = jnp.sum(h * w2_r[...], axis=1, keepdims=True)
        o_r[...] = y + b2_r[...]

    return pl.pallas_call(
        body,
        grid=(1,),
        in_specs=[
            pl.BlockSpec((8, HID), lambda i: (0, 0)),
            pl.BlockSpec((8, HID), lambda i: (0, 0)),
            pl.BlockSpec((8, HID), lambda i: (0, 0)),
            pl.BlockSpec((8, HID), lambda i: (0, 0)),
            pl.BlockSpec((4 * HID, HID), lambda i: (0, 0)),
            pl.BlockSpec((1, HID), lambda i: (0, 0)),
            pl.BlockSpec((1, HID), lambda i: (0, 0)),
            pl.BlockSpec((1, HID), lambda i: (0, 0)),
        ],
        out_specs=pl.BlockSpec((1, HID), lambda i: (0, 0)),
        out_shape=jax.ShapeDtypeStruct((1, HID), _f32),
    )(csb, csk, cs2, cs3, w1, b1, w2t, b2p)


def _edges3(e_arr, e):
    src = e_arr[0].astype(jnp.int32).reshape(NWORK, e // NWORK // CH, CH)
    dst = e_arr[1].astype(jnp.int32).reshape(NWORK, e // NWORK // CH, CH)
    return src, dst


def _gin_tc(x, a_p, p, pfx, n, d):
    w1 = p[pfx + "_W1"]
    if w1.shape[0] < d:
        w1 = jnp.pad(w1, ((0, d - w1.shape[0]), (0, 0)))
    h1, st = _tc_a2_gen(x, a_p, w1, p[pfx + "_b1"].reshape(1, HID), n, d)
    return _tc_a3(h1, st, p[pfx + "_g1"].reshape(1, HID),
                  p[pfx + "_bt1"].reshape(1, HID), p[pfx + "_W2"],
                  p[pfx + "_b2"].reshape(1, HID), n)


def kernel(feats_A, pca_C2, pca_C3, params, edge_bond, edge_knn,
           g1_dst, edge_I2, g2_dst, edge_I3):
    p = params
    feats = feats_A.astype(_f32)

    rep4 = jnp.repeat(jnp.arange(HEADS), HID // HEADS)
    r16 = jnp.zeros((16, HID), _f32).at[rep4, jnp.arange(HID)].set(1.0)
    b16 = jnp.concatenate([jnp.zeros((4,), _f32),
                           jnp.full((12,), -1e30, _f32)]).reshape(1, 16)
    zs = jnp.zeros((NA, HID), _f32)
    zden = jnp.zeros((NA8, HID), _f32)

    hs = []
    colsums = []
    nchg = EB // NWORK // CHG
    for et, ei in (("bond", edge_bond), ("knn", edge_knn)):
        src3, dst3 = _edges3(ei, EB)
        src3g = src3.reshape(NWORK, nchg, CHG)
        dst3g = dst3.reshape(NWORK, nchg, CHG)
        al16 = jnp.zeros((HID, 16), _f32).at[jnp.arange(HID), rep4].set(
            p[et + "_al"].reshape(-1))
        ar16 = jnp.zeros((HID, 16), _f32).at[jnp.arange(HID), rep4].set(
            p[et + "_ar"].reshape(-1))
        g_tbl, er_tbl = _tc_zelter(feats, p[et + "_Wg"], al16, ar16, b16)
        d8_3 = dst3g // 8
        s_p, ex_e = _sc_gat(g_tbl, er_tbl, src3g, dst3g, zs)
        den_p = _sc_den(ex_e.reshape(NWORK, nchg, CHG, 16), dst3g, d8_3, zden)
        gat = _tc_gat_finish(s_p, den_p.reshape(NCORE, NA, 16), r16)
        az = _sc_seg_sum(gat.reshape(1, NA, HID), src3, dst3, zs,
                         NA, HID, EB)
        ax = _sc_seg_sum(feats.reshape(1, NA, IN), src3, dst3, zs,
                         NA, IN, EB)
        azax = jnp.concatenate([az, ax], axis=1)
        h1, st = _tc_a2_atoms(gat, feats, azax,
                              p[et + "_W1"][:HID], p[et + "_W1"][HID:],
                              p[et + "_b1"].reshape(1, HID))
        h_et, cs = _tc_a3(h1, st, p[et + "_g1"].reshape(1, HID),
                          p[et + "_bt1"].reshape(1, HID), p[et + "_W2"],
                          p[et + "_b2"].reshape(1, HID), NA)
        hs.append(h_et)
        colsums.append(cs)

    mb = _sc_seg_max(hs[0], g1_dst, NA, NC2).reshape(
        4, 8, NC2, 16).transpose(0, 2, 1, 3).reshape(4, NC2, 128)
    mk = _sc_seg_max(hs[1], g1_dst, NA, NC2).reshape(
        4, 8, NC2, 16).transpose(0, 2, 1, 3).reshape(4, NC2, 128)
    pca2 = pca_C2.astype(_f32).reshape(NC2, 16)
    h2cat = _tc_combine([mb, mk], pca2, NC2, 3 * HID)

    i2s, i2d = _edges3(edge_I2, EI2)
    z2 = jnp.zeros((NC2, HID), _f32)
    a_p = _sc_seg_sum(h2cat, i2s, i2d, z2, NC2, 3 * HID, EI2)
    h2g1, _ = _gin_tc(h2cat, a_p, p, "h2_0", NC2, 3 * HID)
    h2g1g = h2g1.reshape(1, NC2, HID)
    a_p = _sc_seg_sum(h2g1g, i2s, i2d, z2, NC2, HID, EI2)
    h2g2, cs2 = _gin_tc(h2g1g, a_p, p, "h2_1", NC2, HID)

    m3 = _sc_seg_max(h2g2, g2_dst, NC2, NC3).reshape(
        4, 8, NC3, 16).transpose(0, 2, 1, 3).reshape(4, NC3, 128)
    pca3 = pca_C3.astype(_f32).reshape(NC3, 16)
    h3cat = _tc_combine([m3], pca3, NC3, 2 * HID)

    i3s, i3d = _edges3(edge_I3, EI3)
    z3 = jnp.zeros((NC3, HID), _f32)
    a_p = _sc_seg_sum(h3cat, i3s, i3d, z3, NC3, 2 * HID, EI3)
    h3g1, _ = _gin_tc(h3cat, a_p, p, "h3_0", NC3, 2 * HID)
    h3g1g = h3g1.reshape(1, NC3, HID)
    a_p = _sc_seg_sum(h3g1g, i3s, i3d, z3, NC3, HID, EI3)
    h3g2, cs3 = _gin_tc(h3g1g, a_p, p, "h3_1", NC3, HID)

    b2p = jnp.zeros((1, HID), _f32).at[0, 0].set(p["out_b2"][0])
    y = _tc_readout(colsums[0], colsums[1], cs2, cs3,
                    p["out_W1"], p["out_b1"].reshape(1, HID),
                    p["out_W2"].reshape(1, HID), b2p)
    return y[0:1, 0:1]
